# baseline (speedup 1.0000x reference)
import functools

import jax
import jax.numpy as jnp
from jax import lax
from jax.experimental import pallas as pl
from jax.experimental.pallas import tpu as pltpu
from jax.experimental.pallas import tpu_sc as plsc

N_ATOMS = 524288
N_MOL = 8192
FEAT = 64
MAX_Z = 100
KDIM = 104
BN = 1024
NBN = N_ATOMS // BN

NC = 2
NS = 16
NW = NC * NS
CHUNK = N_ATOMS // NW
BLKG = 2048
NBLKG = CHUNK // BLKG


def _table_body(emb_ref, w_ref, wl_ref, b_ref, out_ref):
    w1 = w_ref[...][:, :FEAT]
    acc = lax.dot_general(
        emb_ref[...], w1, (((1,), (1,)), ((), ())),
        preferred_element_type=jnp.float32) + b_ref[...]
    ii = lax.broadcasted_iota(jnp.int32, (KDIM, FEAT), 0)
    out_ref[...] = jnp.where(
        ii < MAX_Z, acc, jnp.where(ii == MAX_Z, wl_ref[...], 0.0))


def _fused_table(emb104, w, wl2d, b2d):
    return pl.pallas_call(
        _table_body,
        out_shape=jax.ShapeDtypeStruct((KDIM, FEAT), jnp.float32),
    )(emb104, w, wl2d, b2d)


def _featurize_body(z_ref, c_ref, t_ref, wl_ref, out_ref):
    zb = z_ref[...][None, :]
    ii = lax.broadcasted_iota(jnp.int32, (KDIM, BN), 0)
    mt = jnp.where(ii == zb, 1.0, 0.0)
    out = lax.dot_general(
        mt, t_ref[...], (((0,), (0,)), ((), ())),
        preferred_element_type=jnp.float32)
    out_ref[...] = out + c_ref[...] * wl_ref[...]


def _featurize_tc(z, c_exp2d, tbl, wl2d):
    return pl.pallas_call(
        _featurize_body,
        grid=(NBN,),
        in_specs=[
            pl.BlockSpec((BN,), lambda i: (i,)),
            pl.BlockSpec((BN, 1), lambda i: (i, 0)),
            pl.BlockSpec((KDIM, FEAT), lambda i: (0, 0)),
            pl.BlockSpec((1, FEAT), lambda i: (0, 0)),
        ],
        out_specs=pl.BlockSpec((BN, FEAT), lambda i: (i, 0)),
        out_shape=jax.ShapeDtypeStruct((N_ATOMS, FEAT), jnp.float32),
    )(z, c_exp2d, tbl, wl2d)


_MESH = plsc.VectorSubcoreMesh(
    core_axis_name="c", subcore_axis_name="s", num_cores=NC, num_subcores=NS)


@functools.partial(
    pl.kernel,
    out_type=jax.ShapeDtypeStruct((N_ATOMS,), jnp.float32),
    mesh=_MESH,
    scratch_types=[
        pltpu.VMEM((BLKG,), jnp.int32),
        pltpu.VMEM((BLKG,), jnp.int32),
        pltpu.VMEM((BLKG,), jnp.float32),
        pltpu.VMEM((BLKG,), jnp.float32),
        pltpu.VMEM_SHARED((N_MOL,), jnp.float32),
        pltpu.SemaphoreType.DMA,
        pltpu.SemaphoreType.DMA,
        pltpu.SemaphoreType.DMA,
        pltpu.SemaphoreType.DMA,
        pltpu.SemaphoreType.DMA,
        pltpu.SemaphoreType.DMA,
    ],
    compiler_params=pltpu.CompilerParams(needs_layout_passes=False),
)
def _sc_expand(s_hbm, chg_hbm, out_hbm,
               sv0, sv1, cv0, cv1, chg_sp,
               semS0, semS1, semG0, semG1, semO0, semO1):
    sv = (sv0, sv1)
    cv = (cv0, cv1)
    semS = (semS0, semS1)
    semG = (semG0, semG1)
    semO = (semO0, semO1)

    tid = lax.axis_index("s")
    wid = tid * NC + lax.axis_index("c")
    base = wid * CHUNK

    @pl.when(tid == 0)
    def _():
        pltpu.sync_copy(chg_hbm, chg_sp)

    plsc.subcore_barrier()

    def issue_s(blk, b):
        pltpu.async_copy(s_hbm.at[pl.ds(base + blk * BLKG, BLKG)], sv[b],
                         semS[b])

    def wait_s(blk, b):
        pltpu.make_async_copy(s_hbm.at[pl.ds(base + blk * BLKG, BLKG)],
                              sv[b], semS[b]).wait()

    issue_s(0, 0)
    issue_s(1, 1)

    @pl.loop(0, NBLKG, step=2)
    def _blocks(blk2):
        for b in range(2):
            blk = blk2 + b
            row = base + blk * BLKG

            wait_s(blk, b)
            pltpu.async_copy(chg_sp.at[sv[b]], cv[b], semG[b])
            pltpu.make_async_copy(chg_sp.at[sv[b]], cv[b], semG[b]).wait()

            @pl.when(blk >= 2)
            def _():
                prow = base + (blk - 2) * BLKG
                pltpu.make_async_copy(
                    cv[b], out_hbm.at[pl.ds(prow, BLKG)], semO[b]).wait()

            pltpu.async_copy(cv[b], out_hbm.at[pl.ds(row, BLKG)], semO[b])

            @pl.when(blk + 2 < NBLKG)
            def _():
                issue_s(blk + 2, b)

    for b in range(2):
        tail = base + (NBLKG - 2 + b) * BLKG
        pltpu.make_async_copy(
            cv[b], out_hbm.at[pl.ds(tail, BLKG)], semO[b]).wait()


def kernel(atomic_numbers, per_system_total_charge, atomic_subsystem_indices,
           emb_table, W, b):
    z = atomic_numbers.astype(jnp.int32)
    s = atomic_subsystem_indices.astype(jnp.int32)
    emb = emb_table.astype(jnp.float32)
    w = W.astype(jnp.float32)
    chg = per_system_total_charge.astype(jnp.float32)

    emb104 = jnp.pad(emb, ((0, KDIM - MAX_Z), (0, 0)))
    wl2d = w[:, FEAT].reshape(1, FEAT)
    tbl = _fused_table(emb104, w, wl2d, b.astype(jnp.float32).reshape(1, FEAT))

    c_exp = _sc_expand(s, chg)
    return _featurize_tc(z, c_exp.reshape(N_ATOMS, 1), tbl, wl2d)


# speedup vs baseline: 1.3313x; 1.3313x over previous
import functools

import jax
import jax.numpy as jnp
from jax import lax
from jax.experimental import pallas as pl
from jax.experimental.pallas import tpu as pltpu
from jax.experimental.pallas import tpu_sc as plsc

N_ATOMS = 524288
N_MOL = 8192
FEAT = 64
MAX_Z = 100
KDIM = 104
BN = 1024
NBN = N_ATOMS // BN

NC = 2
NS = 16
NW = NC * NS
CHUNK = N_ATOMS // NW
BLKG = 2048
NBLKG = CHUNK // BLKG


def _table_body(emb_ref, w_ref, wl_ref, b_ref, out_ref):
    w1 = w_ref[...][:, :FEAT]
    acc = lax.dot_general(
        emb_ref[...], w1, (((1,), (1,)), ((), ())),
        preferred_element_type=jnp.float32) + b_ref[...]
    ii = lax.broadcasted_iota(jnp.int32, (KDIM, FEAT), 0)
    out_ref[...] = jnp.where(
        ii < MAX_Z, acc, jnp.where(ii == MAX_Z, wl_ref[...], 0.0))


def _fused_table(emb104, w, wl2d, b2d):
    return pl.pallas_call(
        _table_body,
        out_shape=jax.ShapeDtypeStruct((KDIM, FEAT), jnp.float32),
    )(emb104, w, wl2d, b2d)


def _featurize_body(z_ref, c_ref, t_ref, out_ref):
    zb = z_ref[...][None, :]
    cb = c_ref[...][None, :]
    ii = lax.broadcasted_iota(jnp.int32, (KDIM, BN), 0)
    mtf = jnp.where(ii == zb, 1.0, 0.0) + jnp.where(ii == MAX_Z, cb, 0.0)
    mt = mtf.astype(jnp.bfloat16)
    out_t = lax.dot_general(
        t_ref[...].astype(jnp.bfloat16), mt, (((0,), (0,)), ((), ())),
        preferred_element_type=jnp.float32)
    out_ref[...] = out_t.T


def _featurize_tc(z, c_exp, tbl):
    return pl.pallas_call(
        _featurize_body,
        grid=(NBN,),
        in_specs=[
            pl.BlockSpec((BN,), lambda i: (i,)),
            pl.BlockSpec((BN,), lambda i: (i,)),
            pl.BlockSpec((KDIM, FEAT), lambda i: (0, 0)),
        ],
        out_specs=pl.BlockSpec((BN, FEAT), lambda i: (i, 0)),
        out_shape=jax.ShapeDtypeStruct((N_ATOMS, FEAT), jnp.float32),
    )(z, c_exp, tbl)


_MESH = plsc.VectorSubcoreMesh(
    core_axis_name="c", subcore_axis_name="s", num_cores=NC, num_subcores=NS)


@functools.partial(
    pl.kernel,
    out_type=jax.ShapeDtypeStruct((N_ATOMS,), jnp.float32),
    mesh=_MESH,
    scratch_types=[
        pltpu.VMEM((BLKG,), jnp.int32),
        pltpu.VMEM((BLKG,), jnp.int32),
        pltpu.VMEM((BLKG,), jnp.float32),
        pltpu.VMEM((BLKG,), jnp.float32),
        pltpu.VMEM_SHARED((N_MOL,), jnp.float32),
        pltpu.SemaphoreType.DMA,
        pltpu.SemaphoreType.DMA,
        pltpu.SemaphoreType.DMA,
        pltpu.SemaphoreType.DMA,
        pltpu.SemaphoreType.DMA,
        pltpu.SemaphoreType.DMA,
    ],
    compiler_params=pltpu.CompilerParams(needs_layout_passes=False),
)
def _sc_expand(s_hbm, chg_hbm, out_hbm,
               sv0, sv1, cv0, cv1, chg_sp,
               semS0, semS1, semG0, semG1, semO0, semO1):
    sv = (sv0, sv1)
    cv = (cv0, cv1)
    semS = (semS0, semS1)
    semG = (semG0, semG1)
    semO = (semO0, semO1)

    tid = lax.axis_index("s")
    wid = tid * NC + lax.axis_index("c")
    base = wid * CHUNK

    @pl.when(tid == 0)
    def _():
        pltpu.sync_copy(chg_hbm, chg_sp)

    plsc.subcore_barrier()

    def issue_s(blk, b):
        pltpu.async_copy(s_hbm.at[pl.ds(base + blk * BLKG, BLKG)], sv[b],
                         semS[b])

    def wait_s(blk, b):
        pltpu.make_async_copy(s_hbm.at[pl.ds(base + blk * BLKG, BLKG)],
                              sv[b], semS[b]).wait()

    issue_s(0, 0)
    issue_s(1, 1)

    @pl.loop(0, NBLKG, step=2)
    def _blocks(blk2):
        for b in range(2):
            blk = blk2 + b
            row = base + blk * BLKG

            wait_s(blk, b)
            pltpu.async_copy(chg_sp.at[sv[b]], cv[b], semG[b])
            pltpu.make_async_copy(chg_sp.at[sv[b]], cv[b], semG[b]).wait()

            @pl.when(blk >= 2)
            def _():
                prow = base + (blk - 2) * BLKG
                pltpu.make_async_copy(
                    cv[b], out_hbm.at[pl.ds(prow, BLKG)], semO[b]).wait()

            pltpu.async_copy(cv[b], out_hbm.at[pl.ds(row, BLKG)], semO[b])

            @pl.when(blk + 2 < NBLKG)
            def _():
                issue_s(blk + 2, b)

    for b in range(2):
        tail = base + (NBLKG - 2 + b) * BLKG
        pltpu.make_async_copy(
            cv[b], out_hbm.at[pl.ds(tail, BLKG)], semO[b]).wait()


def kernel(atomic_numbers, per_system_total_charge, atomic_subsystem_indices,
           emb_table, W, b):
    z = atomic_numbers.astype(jnp.int32)
    s = atomic_subsystem_indices.astype(jnp.int32)
    emb = emb_table.astype(jnp.float32)
    w = W.astype(jnp.float32)
    chg = per_system_total_charge.astype(jnp.float32)

    emb104 = jnp.pad(emb, ((0, KDIM - MAX_Z), (0, 0)))
    wl2d = w[:, FEAT].reshape(1, FEAT)
    tbl = _fused_table(emb104, w, wl2d, b.astype(jnp.float32).reshape(1, FEAT))

    c_exp = _sc_expand(s, chg)
    return _featurize_tc(z, c_exp, tbl)


# speedup vs baseline: 1.7463x; 1.3118x over previous
import functools

import jax
import jax.numpy as jnp
from jax import lax
from jax.experimental import pallas as pl
from jax.experimental.pallas import tpu as pltpu
from jax.experimental.pallas import tpu_sc as plsc

N_ATOMS = 524288
N_MOL = 8192
FEAT = 64
MAX_Z = 100
KDIM = 104
BN = 2048
NBN = N_ATOMS // BN

NC = 2
NS = 16
NW = NC * NS
CHUNK = N_ATOMS // NW
BLKG = 2048
NBLKG = CHUNK // BLKG


def _table_body(emb_ref, w_ref, wl_ref, b_ref, out_ref):
    w1 = w_ref[...][:, :FEAT]
    acc = lax.dot_general(
        emb_ref[...], w1, (((1,), (1,)), ((), ())),
        preferred_element_type=jnp.float32) + b_ref[...]
    ii = lax.broadcasted_iota(jnp.int32, (KDIM, FEAT), 0)
    out_ref[...] = jnp.where(
        ii < MAX_Z, acc, jnp.where(ii == MAX_Z, wl_ref[...], 0.0))


def _fused_table(emb104, w, wl2d, b2d):
    return pl.pallas_call(
        _table_body,
        out_shape=jax.ShapeDtypeStruct((KDIM, FEAT), jnp.float32),
    )(emb104, w, wl2d, b2d)


def _featurize_body(z_ref, c_ref, t_ref, out_ref):
    zb = z_ref[...][None, :]
    cb = c_ref[...][None, :]
    ii = lax.broadcasted_iota(jnp.int32, (KDIM, BN), 0)
    mtf = jnp.where(ii == zb, 1.0, 0.0) + jnp.where(ii == MAX_Z, cb, 0.0)
    mt = mtf.astype(jnp.bfloat16)
    out_t = lax.dot_general(
        t_ref[...].astype(jnp.bfloat16), mt, (((0,), (0,)), ((), ())),
        preferred_element_type=jnp.float32)
    out_ref[...] = out_t.T


def _featurize_tc(z, c_exp, tbl):
    return pl.pallas_call(
        _featurize_body,
        grid=(NBN,),
        in_specs=[
            pl.BlockSpec((BN,), lambda i: (i,)),
            pl.BlockSpec((BN,), lambda i: (i,)),
            pl.BlockSpec((KDIM, FEAT), lambda i: (0, 0)),
        ],
        out_specs=pl.BlockSpec((BN, FEAT), lambda i: (i, 0)),
        out_shape=jax.ShapeDtypeStruct((N_ATOMS, FEAT), jnp.float32),
    )(z, c_exp, tbl)


_MESH = plsc.VectorSubcoreMesh(
    core_axis_name="c", subcore_axis_name="s", num_cores=NC, num_subcores=NS)


@functools.partial(
    pl.kernel,
    out_type=jax.ShapeDtypeStruct((N_ATOMS,), jnp.float32),
    mesh=_MESH,
    scratch_types=[
        pltpu.VMEM((BLKG,), jnp.int32),
        pltpu.VMEM((BLKG,), jnp.int32),
        pltpu.VMEM((BLKG,), jnp.float32),
        pltpu.VMEM((BLKG,), jnp.float32),
        pltpu.VMEM_SHARED((N_MOL,), jnp.float32),
        pltpu.SemaphoreType.DMA,
        pltpu.SemaphoreType.DMA,
        pltpu.SemaphoreType.DMA,
        pltpu.SemaphoreType.DMA,
        pltpu.SemaphoreType.DMA,
        pltpu.SemaphoreType.DMA,
    ],
    compiler_params=pltpu.CompilerParams(needs_layout_passes=False),
)
def _sc_expand(s_hbm, chg_hbm, out_hbm,
               sv0, sv1, cv0, cv1, chg_sp,
               semS0, semS1, semG0, semG1, semO0, semO1):
    sv = (sv0, sv1)
    cv = (cv0, cv1)
    semS = (semS0, semS1)
    semG = (semG0, semG1)
    semO = (semO0, semO1)

    tid = lax.axis_index("s")
    wid = tid * NC + lax.axis_index("c")
    base = wid * CHUNK

    @pl.when(tid == 0)
    def _():
        pltpu.sync_copy(chg_hbm, chg_sp)

    plsc.subcore_barrier()

    def issue_s(blk, b):
        pltpu.async_copy(s_hbm.at[pl.ds(base + blk * BLKG, BLKG)], sv[b],
                         semS[b])

    def wait_s(blk, b):
        pltpu.make_async_copy(s_hbm.at[pl.ds(base + blk * BLKG, BLKG)],
                              sv[b], semS[b]).wait()

    issue_s(0, 0)
    issue_s(1, 1)

    @pl.loop(0, NBLKG, step=2)
    def _blocks(blk2):
        for b in range(2):
            blk = blk2 + b
            row = base + blk * BLKG

            wait_s(blk, b)
            pltpu.async_copy(chg_sp.at[sv[b]], cv[b], semG[b])
            pltpu.make_async_copy(chg_sp.at[sv[b]], cv[b], semG[b]).wait()

            @pl.when(blk >= 2)
            def _():
                prow = base + (blk - 2) * BLKG
                pltpu.make_async_copy(
                    cv[b], out_hbm.at[pl.ds(prow, BLKG)], semO[b]).wait()

            pltpu.async_copy(cv[b], out_hbm.at[pl.ds(row, BLKG)], semO[b])

            @pl.when(blk + 2 < NBLKG)
            def _():
                issue_s(blk + 2, b)

    for b in range(2):
        tail = base + (NBLKG - 2 + b) * BLKG
        pltpu.make_async_copy(
            cv[b], out_hbm.at[pl.ds(tail, BLKG)], semO[b]).wait()


def kernel(atomic_numbers, per_system_total_charge, atomic_subsystem_indices,
           emb_table, W, b):
    z = atomic_numbers.astype(jnp.int32)
    s = atomic_subsystem_indices.astype(jnp.int32)
    emb = emb_table.astype(jnp.float32)
    w = W.astype(jnp.float32)
    chg = per_system_total_charge.astype(jnp.float32)

    emb104 = jnp.pad(emb, ((0, KDIM - MAX_Z), (0, 0)))
    wl2d = w[:, FEAT].reshape(1, FEAT)
    tbl = _fused_table(emb104, w, wl2d, b.astype(jnp.float32).reshape(1, FEAT))

    c_exp = _sc_expand(s, chg)
    return _featurize_tc(z, c_exp, tbl)


# speedup vs baseline: 2.0866x; 1.1949x over previous
import functools

import jax
import jax.numpy as jnp
from jax import lax
from jax.experimental import pallas as pl
from jax.experimental.pallas import tpu as pltpu
from jax.experimental.pallas import tpu_sc as plsc

N_ATOMS = 524288
N_MOL = 8192
FEAT = 64
MAX_Z = 100
KDIM = 104
BN = 4096
NBN = N_ATOMS // BN

NC = 2
NS = 16
NW = NC * NS
CHUNK = N_ATOMS // NW
BLKG = 2048
NBLKG = CHUNK // BLKG


def _table_body(emb_ref, w_ref, wl_ref, b_ref, out_ref):
    w1 = w_ref[...][:, :FEAT]
    acc = lax.dot_general(
        emb_ref[...], w1, (((1,), (1,)), ((), ())),
        preferred_element_type=jnp.float32) + b_ref[...]
    ii = lax.broadcasted_iota(jnp.int32, (KDIM, FEAT), 0)
    out_ref[...] = jnp.where(
        ii < MAX_Z, acc, jnp.where(ii == MAX_Z, wl_ref[...], 0.0))


def _fused_table(emb104, w, wl2d, b2d):
    return pl.pallas_call(
        _table_body,
        out_shape=jax.ShapeDtypeStruct((KDIM, FEAT), jnp.float32),
    )(emb104, w, wl2d, b2d)


def _featurize_body(z_ref, c_ref, t_ref, out_ref):
    zb = z_ref[...][None, :]
    cb = c_ref[...][None, :]
    ii = lax.broadcasted_iota(jnp.int32, (KDIM, BN), 0)
    mtf = jnp.where(ii == zb, 1.0, 0.0) + jnp.where(ii == MAX_Z, cb, 0.0)
    mt = mtf.astype(jnp.bfloat16)
    out_t = lax.dot_general(
        t_ref[...].astype(jnp.bfloat16), mt, (((0,), (0,)), ((), ())),
        preferred_element_type=jnp.float32)
    out_ref[...] = out_t.T


def _featurize_tc(z, c_exp, tbl):
    return pl.pallas_call(
        _featurize_body,
        grid=(NBN,),
        in_specs=[
            pl.BlockSpec((BN,), lambda i: (i,)),
            pl.BlockSpec((BN,), lambda i: (i,)),
            pl.BlockSpec((KDIM, FEAT), lambda i: (0, 0)),
        ],
        out_specs=pl.BlockSpec((BN, FEAT), lambda i: (i, 0)),
        out_shape=jax.ShapeDtypeStruct((N_ATOMS, FEAT), jnp.float32),
    )(z, c_exp, tbl)


_MESH = plsc.VectorSubcoreMesh(
    core_axis_name="c", subcore_axis_name="s", num_cores=NC, num_subcores=NS)


@functools.partial(
    pl.kernel,
    out_type=jax.ShapeDtypeStruct((N_ATOMS,), jnp.float32),
    mesh=_MESH,
    scratch_types=[
        pltpu.VMEM((BLKG,), jnp.int32),
        pltpu.VMEM((BLKG,), jnp.int32),
        pltpu.VMEM((BLKG,), jnp.float32),
        pltpu.VMEM((BLKG,), jnp.float32),
        pltpu.VMEM_SHARED((N_MOL,), jnp.float32),
        pltpu.SemaphoreType.DMA,
        pltpu.SemaphoreType.DMA,
        pltpu.SemaphoreType.DMA,
        pltpu.SemaphoreType.DMA,
        pltpu.SemaphoreType.DMA,
        pltpu.SemaphoreType.DMA,
    ],
    compiler_params=pltpu.CompilerParams(needs_layout_passes=False),
)
def _sc_expand(s_hbm, chg_hbm, out_hbm,
               sv0, sv1, cv0, cv1, chg_sp,
               semS0, semS1, semG0, semG1, semO0, semO1):
    sv = (sv0, sv1)
    cv = (cv0, cv1)
    semS = (semS0, semS1)
    semG = (semG0, semG1)
    semO = (semO0, semO1)

    tid = lax.axis_index("s")
    wid = tid * NC + lax.axis_index("c")
    base = wid * CHUNK

    @pl.when(tid == 0)
    def _():
        pltpu.sync_copy(chg_hbm, chg_sp)

    plsc.subcore_barrier()

    def issue_s(blk, b):
        pltpu.async_copy(s_hbm.at[pl.ds(base + blk * BLKG, BLKG)], sv[b],
                         semS[b])

    def wait_s(blk, b):
        pltpu.make_async_copy(s_hbm.at[pl.ds(base + blk * BLKG, BLKG)],
                              sv[b], semS[b]).wait()

    issue_s(0, 0)
    issue_s(1, 1)

    @pl.loop(0, NBLKG, step=2)
    def _blocks(blk2):
        for b in range(2):
            blk = blk2 + b
            row = base + blk * BLKG

            wait_s(blk, b)
            pltpu.async_copy(chg_sp.at[sv[b]], cv[b], semG[b])
            pltpu.make_async_copy(chg_sp.at[sv[b]], cv[b], semG[b]).wait()

            @pl.when(blk >= 2)
            def _():
                prow = base + (blk - 2) * BLKG
                pltpu.make_async_copy(
                    cv[b], out_hbm.at[pl.ds(prow, BLKG)], semO[b]).wait()

            pltpu.async_copy(cv[b], out_hbm.at[pl.ds(row, BLKG)], semO[b])

            @pl.when(blk + 2 < NBLKG)
            def _():
                issue_s(blk + 2, b)

    for b in range(2):
        tail = base + (NBLKG - 2 + b) * BLKG
        pltpu.make_async_copy(
            cv[b], out_hbm.at[pl.ds(tail, BLKG)], semO[b]).wait()


def kernel(atomic_numbers, per_system_total_charge, atomic_subsystem_indices,
           emb_table, W, b):
    z = atomic_numbers.astype(jnp.int32)
    s = atomic_subsystem_indices.astype(jnp.int32)
    emb = emb_table.astype(jnp.float32)
    w = W.astype(jnp.float32)
    chg = per_system_total_charge.astype(jnp.float32)

    emb104 = jnp.pad(emb, ((0, KDIM - MAX_Z), (0, 0)))
    wl2d = w[:, FEAT].reshape(1, FEAT)
    tbl = _fused_table(emb104, w, wl2d, b.astype(jnp.float32).reshape(1, FEAT))

    c_exp = _sc_expand(s, chg)
    return _featurize_tc(z, c_exp, tbl)


# speedup vs baseline: 2.3225x; 1.1130x over previous
import functools

import jax
import jax.numpy as jnp
from jax import lax
from jax.experimental import pallas as pl
from jax.experimental.pallas import tpu as pltpu
from jax.experimental.pallas import tpu_sc as plsc

N_ATOMS = 524288
N_MOL = 8192
FEAT = 64
MAX_Z = 100
KDIM = 104
BN = 8192
NBN = N_ATOMS // BN

NC = 2
NS = 16
NW = NC * NS
CHUNK = N_ATOMS // NW
BLKG = 2048
NBLKG = CHUNK // BLKG


def _table_body(emb_ref, w_ref, wl_ref, b_ref, out_ref):
    w1 = w_ref[...][:, :FEAT]
    acc = lax.dot_general(
        emb_ref[...], w1, (((1,), (1,)), ((), ())),
        preferred_element_type=jnp.float32) + b_ref[...]
    ii = lax.broadcasted_iota(jnp.int32, (KDIM, FEAT), 0)
    out_ref[...] = jnp.where(
        ii < MAX_Z, acc, jnp.where(ii == MAX_Z, wl_ref[...], 0.0))


def _fused_table(emb104, w, wl2d, b2d):
    return pl.pallas_call(
        _table_body,
        out_shape=jax.ShapeDtypeStruct((KDIM, FEAT), jnp.float32),
    )(emb104, w, wl2d, b2d)


def _featurize_body(z_ref, c_ref, t_ref, out_ref):
    zb = z_ref[...][None, :]
    cb = c_ref[...][None, :]
    ii = lax.broadcasted_iota(jnp.int32, (KDIM, BN), 0)
    mtf = jnp.where(ii == zb, 1.0, 0.0) + jnp.where(ii == MAX_Z, cb, 0.0)
    mt = mtf.astype(jnp.bfloat16)
    out_t = lax.dot_general(
        t_ref[...].astype(jnp.bfloat16), mt, (((0,), (0,)), ((), ())),
        preferred_element_type=jnp.float32)
    out_ref[...] = out_t.T


def _featurize_tc(z, c_exp, tbl):
    return pl.pallas_call(
        _featurize_body,
        grid=(NBN,),
        in_specs=[
            pl.BlockSpec((BN,), lambda i: (i,)),
            pl.BlockSpec((BN,), lambda i: (i,)),
            pl.BlockSpec((KDIM, FEAT), lambda i: (0, 0)),
        ],
        out_specs=pl.BlockSpec((BN, FEAT), lambda i: (i, 0)),
        out_shape=jax.ShapeDtypeStruct((N_ATOMS, FEAT), jnp.float32),
    )(z, c_exp, tbl)


_MESH = plsc.VectorSubcoreMesh(
    core_axis_name="c", subcore_axis_name="s", num_cores=NC, num_subcores=NS)


@functools.partial(
    pl.kernel,
    out_type=jax.ShapeDtypeStruct((N_ATOMS,), jnp.float32),
    mesh=_MESH,
    scratch_types=[
        pltpu.VMEM((BLKG,), jnp.int32),
        pltpu.VMEM((BLKG,), jnp.int32),
        pltpu.VMEM((BLKG,), jnp.float32),
        pltpu.VMEM((BLKG,), jnp.float32),
        pltpu.VMEM_SHARED((N_MOL,), jnp.float32),
        pltpu.SemaphoreType.DMA,
        pltpu.SemaphoreType.DMA,
        pltpu.SemaphoreType.DMA,
        pltpu.SemaphoreType.DMA,
        pltpu.SemaphoreType.DMA,
        pltpu.SemaphoreType.DMA,
    ],
    compiler_params=pltpu.CompilerParams(needs_layout_passes=False),
)
def _sc_expand(s_hbm, chg_hbm, out_hbm,
               sv0, sv1, cv0, cv1, chg_sp,
               semS0, semS1, semG0, semG1, semO0, semO1):
    sv = (sv0, sv1)
    cv = (cv0, cv1)
    semS = (semS0, semS1)
    semG = (semG0, semG1)
    semO = (semO0, semO1)

    tid = lax.axis_index("s")
    wid = tid * NC + lax.axis_index("c")
    base = wid * CHUNK

    @pl.when(tid == 0)
    def _():
        pltpu.sync_copy(chg_hbm, chg_sp)

    plsc.subcore_barrier()

    def issue_s(blk, b):
        pltpu.async_copy(s_hbm.at[pl.ds(base + blk * BLKG, BLKG)], sv[b],
                         semS[b])

    def wait_s(blk, b):
        pltpu.make_async_copy(s_hbm.at[pl.ds(base + blk * BLKG, BLKG)],
                              sv[b], semS[b]).wait()

    issue_s(0, 0)
    issue_s(1, 1)

    @pl.loop(0, NBLKG, step=2)
    def _blocks(blk2):
        for b in range(2):
            blk = blk2 + b
            row = base + blk * BLKG

            wait_s(blk, b)
            pltpu.async_copy(chg_sp.at[sv[b]], cv[b], semG[b])
            pltpu.make_async_copy(chg_sp.at[sv[b]], cv[b], semG[b]).wait()

            @pl.when(blk >= 2)
            def _():
                prow = base + (blk - 2) * BLKG
                pltpu.make_async_copy(
                    cv[b], out_hbm.at[pl.ds(prow, BLKG)], semO[b]).wait()

            pltpu.async_copy(cv[b], out_hbm.at[pl.ds(row, BLKG)], semO[b])

            @pl.when(blk + 2 < NBLKG)
            def _():
                issue_s(blk + 2, b)

    for b in range(2):
        tail = base + (NBLKG - 2 + b) * BLKG
        pltpu.make_async_copy(
            cv[b], out_hbm.at[pl.ds(tail, BLKG)], semO[b]).wait()


def kernel(atomic_numbers, per_system_total_charge, atomic_subsystem_indices,
           emb_table, W, b):
    z = atomic_numbers.astype(jnp.int32)
    s = atomic_subsystem_indices.astype(jnp.int32)
    emb = emb_table.astype(jnp.float32)
    w = W.astype(jnp.float32)
    chg = per_system_total_charge.astype(jnp.float32)

    emb104 = jnp.pad(emb, ((0, KDIM - MAX_Z), (0, 0)))
    wl2d = w[:, FEAT].reshape(1, FEAT)
    tbl = _fused_table(emb104, w, wl2d, b.astype(jnp.float32).reshape(1, FEAT))

    c_exp = _sc_expand(s, chg)
    return _featurize_tc(z, c_exp, tbl)


# speedup vs baseline: 2.4171x; 1.0407x over previous
import functools

import jax
import jax.numpy as jnp
from jax import lax
from jax.experimental import pallas as pl
from jax.experimental.pallas import tpu as pltpu
from jax.experimental.pallas import tpu_sc as plsc

N_ATOMS = 524288
N_MOL = 8192
FEAT = 64
MAX_Z = 100
KDIM = 104
BN = 16384
NBN = N_ATOMS // BN

NC = 2
NS = 16
NW = NC * NS
CHUNK = N_ATOMS // NW
BLKG = 2048
NBLKG = CHUNK // BLKG


def _table_body(emb_ref, w_ref, wl_ref, b_ref, out_ref):
    w1 = w_ref[...][:, :FEAT]
    acc = lax.dot_general(
        emb_ref[...], w1, (((1,), (1,)), ((), ())),
        preferred_element_type=jnp.float32) + b_ref[...]
    ii = lax.broadcasted_iota(jnp.int32, (KDIM, FEAT), 0)
    out_ref[...] = jnp.where(
        ii < MAX_Z, acc, jnp.where(ii == MAX_Z, wl_ref[...], 0.0))


def _fused_table(emb104, w, wl2d, b2d):
    return pl.pallas_call(
        _table_body,
        out_shape=jax.ShapeDtypeStruct((KDIM, FEAT), jnp.float32),
    )(emb104, w, wl2d, b2d)


def _featurize_body(z_ref, c_ref, t_ref, out_ref):
    zb = z_ref[...][None, :]
    cb = c_ref[...][None, :]
    ii = lax.broadcasted_iota(jnp.int32, (KDIM, BN), 0)
    mtf = jnp.where(ii == zb, 1.0, 0.0) + jnp.where(ii == MAX_Z, cb, 0.0)
    mt = mtf.astype(jnp.bfloat16)
    out_t = lax.dot_general(
        t_ref[...].astype(jnp.bfloat16), mt, (((0,), (0,)), ((), ())),
        preferred_element_type=jnp.float32)
    out_ref[...] = out_t.T


def _featurize_tc(z, c_exp, tbl):
    return pl.pallas_call(
        _featurize_body,
        grid=(NBN,),
        in_specs=[
            pl.BlockSpec((BN,), lambda i: (i,)),
            pl.BlockSpec((BN,), lambda i: (i,)),
            pl.BlockSpec((KDIM, FEAT), lambda i: (0, 0)),
        ],
        out_specs=pl.BlockSpec((BN, FEAT), lambda i: (i, 0)),
        out_shape=jax.ShapeDtypeStruct((N_ATOMS, FEAT), jnp.float32),
    )(z, c_exp, tbl)


_MESH = plsc.VectorSubcoreMesh(
    core_axis_name="c", subcore_axis_name="s", num_cores=NC, num_subcores=NS)


@functools.partial(
    pl.kernel,
    out_type=jax.ShapeDtypeStruct((N_ATOMS,), jnp.float32),
    mesh=_MESH,
    scratch_types=[
        pltpu.VMEM((BLKG,), jnp.int32),
        pltpu.VMEM((BLKG,), jnp.int32),
        pltpu.VMEM((BLKG,), jnp.float32),
        pltpu.VMEM((BLKG,), jnp.float32),
        pltpu.VMEM_SHARED((N_MOL,), jnp.float32),
        pltpu.SemaphoreType.DMA,
        pltpu.SemaphoreType.DMA,
        pltpu.SemaphoreType.DMA,
        pltpu.SemaphoreType.DMA,
        pltpu.SemaphoreType.DMA,
        pltpu.SemaphoreType.DMA,
    ],
    compiler_params=pltpu.CompilerParams(needs_layout_passes=False),
)
def _sc_expand(s_hbm, chg_hbm, out_hbm,
               sv0, sv1, cv0, cv1, chg_sp,
               semS0, semS1, semG0, semG1, semO0, semO1):
    sv = (sv0, sv1)
    cv = (cv0, cv1)
    semS = (semS0, semS1)
    semG = (semG0, semG1)
    semO = (semO0, semO1)

    tid = lax.axis_index("s")
    wid = tid * NC + lax.axis_index("c")
    base = wid * CHUNK

    @pl.when(tid == 0)
    def _():
        pltpu.sync_copy(chg_hbm, chg_sp)

    plsc.subcore_barrier()

    def issue_s(blk, b):
        pltpu.async_copy(s_hbm.at[pl.ds(base + blk * BLKG, BLKG)], sv[b],
                         semS[b])

    def wait_s(blk, b):
        pltpu.make_async_copy(s_hbm.at[pl.ds(base + blk * BLKG, BLKG)],
                              sv[b], semS[b]).wait()

    issue_s(0, 0)
    issue_s(1, 1)

    @pl.loop(0, NBLKG, step=2)
    def _blocks(blk2):
        for b in range(2):
            blk = blk2 + b
            row = base + blk * BLKG

            wait_s(blk, b)
            pltpu.async_copy(chg_sp.at[sv[b]], cv[b], semG[b])
            pltpu.make_async_copy(chg_sp.at[sv[b]], cv[b], semG[b]).wait()

            @pl.when(blk >= 2)
            def _():
                prow = base + (blk - 2) * BLKG
                pltpu.make_async_copy(
                    cv[b], out_hbm.at[pl.ds(prow, BLKG)], semO[b]).wait()

            pltpu.async_copy(cv[b], out_hbm.at[pl.ds(row, BLKG)], semO[b])

            @pl.when(blk + 2 < NBLKG)
            def _():
                issue_s(blk + 2, b)

    for b in range(2):
        tail = base + (NBLKG - 2 + b) * BLKG
        pltpu.make_async_copy(
            cv[b], out_hbm.at[pl.ds(tail, BLKG)], semO[b]).wait()


def kernel(atomic_numbers, per_system_total_charge, atomic_subsystem_indices,
           emb_table, W, b):
    z = atomic_numbers.astype(jnp.int32)
    s = atomic_subsystem_indices.astype(jnp.int32)
    emb = emb_table.astype(jnp.float32)
    w = W.astype(jnp.float32)
    chg = per_system_total_charge.astype(jnp.float32)

    emb104 = jnp.pad(emb, ((0, KDIM - MAX_Z), (0, 0)))
    wl2d = w[:, FEAT].reshape(1, FEAT)
    tbl = _fused_table(emb104, w, wl2d, b.astype(jnp.float32).reshape(1, FEAT))

    c_exp = _sc_expand(s, chg)
    return _featurize_tc(z, c_exp, tbl)
